# baseline (speedup 1.0000x reference)
import functools

import jax
import jax.numpy as jnp
from jax import lax
from jax.experimental import pallas as pl
from jax.experimental.pallas import tpu as pltpu
from jax.experimental.pallas import tpu_sc as plsc

_NU = 8000
_NP = 2000
_N = _NU + _NP
_E = 320000
_D = 128

_NCORES = 2
_NSUB = 16
_NTILES = _NCORES * _NSUB
_CHUNK = 128
_NCHUNK = 80
_EPT = _CHUNK * _NCHUNK
_EPAD = _NTILES * _EPT
_NPADT = _N + 16
_RPT = _N // _NSUB


_CPT = 624
_NPADC = _NSUB * 640


@functools.lru_cache(maxsize=None)
def _make_seg_sum(with_cnt):
    mesh = plsc.VectorSubcoreMesh(core_axis_name="c", subcore_axis_name="s",
                                  num_cores=_NCORES, num_subcores=_NSUB)
    out_type = [jax.ShapeDtypeStruct((_NCORES, _NSUB, _RPT, _D), jnp.float32)]
    scratch = [
        pltpu.VMEM((_CHUNK, _D), jnp.float32),
        pltpu.VMEM((_CHUNK, _D), jnp.float32),
        pltpu.VMEM((_NCHUNK, _CHUNK), jnp.int32),
        pltpu.VMEM((_CHUNK,), jnp.int32),
        pltpu.VMEM((_CHUNK,), jnp.int32),
        pltpu.VMEM((_CHUNK,), jnp.int32),
        pltpu.VMEM((_CHUNK,), jnp.int32),
        pltpu.VMEM_SHARED((_NPADT, _D), jnp.float32),
        pltpu.SemaphoreType.DMA,
        pltpu.SemaphoreType.DMA,
    ]
    if with_cnt:
        out_type.append(
            jax.ShapeDtypeStruct((_NCORES, 1, _NPADC), jnp.float32))
        scratch += [
            pltpu.VMEM((_CHUNK,), jnp.float32),
            pltpu.VMEM((640,), jnp.float32),
            pltpu.VMEM_SHARED((_NPADC,), jnp.float32),
        ]

    def body(table, pidx2d, agg_out, *rest):
        if with_cnt:
            cnt_out, gbuf0, gbuf1, pidx, sidx0, didx0, sidx1, didx1, acc, \
                sem0, sem1, onesv, zc, cacc = rest
        else:
            gbuf0, gbuf1, pidx, sidx0, didx0, sidx1, didx1, acc, sem0, \
                sem1 = rest
            cnt_out = cacc = onesv = zc = None
        cid = lax.axis_index("c")
        sid = lax.axis_index("s")
        wid = cid * _NSUB + sid
        r0 = sid * _RPT
        base = wid * _NCHUNK

        pltpu.sync_copy(pidx2d.at[pl.ds(base, _NCHUNK)], pidx)
        zeros16 = jnp.zeros((16,), jnp.float32)

        def unpack(c, sidx, didx):
            for j in range(_CHUNK // 16):
                v = pidx[c, pl.ds(j * 16, 16)]
                sidx[pl.ds(j * 16, 16)] = lax.bitwise_and(v, 0xFFFF)
                didx[pl.ds(j * 16, 16)] = lax.shift_right_logical(v, 16)

        def fill_zrow(i, c):
            gbuf0[i // 8, pl.ds((i % 8) * 16, 16)] = zeros16
            return c

        lax.fori_loop(0, _CHUNK * 8, fill_zrow, 0)
        for k in range(5):
            pltpu.sync_copy(gbuf0.at[pl.ds(0, _RPT // 5)],
                            acc.at[pl.ds(r0 + k * (_RPT // 5), _RPT // 5)])
        if with_cnt:
            ones16 = jnp.full((16,), 1.0, jnp.float32)

            def fill_ones(i, c):
                onesv[pl.ds(i * 16, 16)] = ones16
                return c

            lax.fori_loop(0, _CHUNK // 16, fill_ones, 0)

            def fill_zero(i, c):
                zc[pl.ds(i * 16, 16)] = zeros16
                return c

            lax.fori_loop(0, 40, fill_zero, 0)
            pltpu.sync_copy(zc, cacc.at[pl.ds(sid * _CPT, 640)])
        plsc.subcore_barrier()

        def chunk_body(c, carry):
            unpack(c, sidx0, didx0)
            pltpu.async_copy(table.at[sidx0], gbuf0, sem0).wait()
            pltpu.sync_copy(gbuf0, acc.at[didx0], add=True)
            if with_cnt:
                pltpu.sync_copy(onesv, cacc.at[didx0], add=True)
            return carry

        lax.fori_loop(0, _NCHUNK, chunk_body, 0)
        plsc.subcore_barrier()

        pltpu.sync_copy(acc.at[pl.ds(r0, _RPT)], agg_out.at[cid, sid])
        if with_cnt:
            pltpu.sync_copy(cacc.at[pl.ds(sid * _CPT, 640)], zc)
            pltpu.sync_copy(zc, cnt_out.at[cid, 0, pl.ds(sid * 640, 640)])

    return pl.kernel(body, out_type, mesh=mesh, scratch_types=scratch)


def _seg_sum_cnt(*args):
    return _make_seg_sum(True)(*args)


def _seg_sum(*args):
    res = _make_seg_sum(False)(*args)
    return res[0] if isinstance(res, (list, tuple)) else res


def _tc_embed(x, W, b, block_rows):
    M, K = x.shape
    H = W.shape[1]

    def body(x_ref, w_ref, b_ref, o_ref):
        o_ref[...] = (jnp.dot(x_ref[...], w_ref[...],
                              preferred_element_type=jnp.float32) + b_ref[...])

    return pl.pallas_call(
        body,
        grid=(M // block_rows,),
        in_specs=[
            pl.BlockSpec((block_rows, K), lambda i: (i, 0)),
            pl.BlockSpec((K, H), lambda i: (0, 0)),
            pl.BlockSpec((1, H), lambda i: (0, 0)),
        ],
        out_specs=pl.BlockSpec((block_rows, H), lambda i: (i, 0)),
        out_shape=jax.ShapeDtypeStruct((M, H), jnp.float32),
    )(x, W, b.reshape(1, H))


def _tc_sage(agg, cnt, x, Wl, bl, Wr):
    BR = 1000

    def body(a_ref, c_ref, x_ref, wl_ref, bl_ref, wr_ref, o_ref):
        a = a_ref[0] + a_ref[1]
        c = c_ref[:, 0:1] + c_ref[:, 1:2]
        mean = a / jnp.maximum(c, 1.0)
        o_ref[...] = jnp.maximum(
            jnp.dot(mean, wl_ref[...], preferred_element_type=jnp.float32)
            + bl_ref[...]
            + jnp.dot(x_ref[...], wr_ref[...],
                      preferred_element_type=jnp.float32),
            0.0)

    return pl.pallas_call(
        body,
        grid=(_N // BR,),
        in_specs=[
            pl.BlockSpec((_NCORES, BR, _D), lambda i: (0, i, 0)),
            pl.BlockSpec((BR, _NCORES), lambda i: (i, 0)),
            pl.BlockSpec((BR, _D), lambda i: (i, 0)),
            pl.BlockSpec((_D, _D), lambda i: (0, 0)),
            pl.BlockSpec((1, _D), lambda i: (0, 0)),
            pl.BlockSpec((_D, _D), lambda i: (0, 0)),
        ],
        out_specs=pl.BlockSpec((BR, _D), lambda i: (i, 0)),
        out_shape=jax.ShapeDtypeStruct((_N, _D), jnp.float32),
    )(agg, cnt, x, Wl, bl.reshape(1, _D), Wr)


def _tc_head(x0, x1, x2, W1a, W1b, W1c, b1, W2, b2, Wc, bc, Wt, bt, WT_, bT_,
             Woc, boc, Wot, bot, WoT, boT):
    BR = 1000
    HH = 64

    def body(x0r, x1r, x2r, w1ar, w1br, w1cr, b1r, w2r, b2r, wcr, bcr,
             wtr, btr, wTr, bTr, wocr, bocr, wotr, botr, wTor, bTor,
             ot1, ot0, oT, ht1, ht0):
        dot = lambda a, w: jnp.dot(a, w, preferred_element_type=jnp.float32)
        h = jnp.maximum(dot(x0r[...], w1ar[...]) + dot(x1r[...], w1br[...])
                        + dot(x2r[...], w1cr[...]) + b1r[...], 0.0)
        h = jnp.maximum(dot(h, w2r[...]) + b2r[...], 0.0)
        a_t0 = jnp.maximum(dot(h, wcr[...]) + bcr[...], 0.0)
        a_t1 = jnp.maximum(dot(h, wtr[...]) + btr[...], 0.0)
        a_T = jnp.maximum(dot(h, wTr[...]) + bTr[...], 0.0)
        ht0[...] = a_t0
        ht1[...] = a_t1
        ot0[...] = jnp.maximum(dot(a_t0, wocr[...]) + bocr[...], 0.0)
        ot1[...] = jnp.maximum(dot(a_t1, wotr[...]) + botr[...], 0.0)
        oT[...] = jnp.maximum(dot(a_T, wTor[...]) + bTor[...], 0.0)

    full = lambda s: pl.BlockSpec(s, lambda i: tuple(0 for _ in s))
    row_spec = lambda w: pl.BlockSpec((BR, w), lambda i: (i, 0))
    outs = pl.pallas_call(
        body,
        grid=(_NU // BR,),
        in_specs=[
            row_spec(_D), row_spec(_D), row_spec(_D),
            full((_D, _D)), full((_D, _D)), full((_D, _D)), full((1, _D)),
            full((_D, _D)), full((1, _D)),
            full((_D, HH)), full((1, HH)),
            full((_D, HH)), full((1, HH)),
            full((_D, HH)), full((1, HH)),
            full((HH, _D)), full((1, _D)),
            full((HH, _D)), full((1, _D)),
            full((HH, _D)), full((1, _D)),
        ],
        out_specs=[
            row_spec(_D), row_spec(_D), row_spec(_D),
            row_spec(HH), row_spec(HH),
        ],
        out_shape=[
            jax.ShapeDtypeStruct((_NU, _D), jnp.float32),
            jax.ShapeDtypeStruct((_NU, _D), jnp.float32),
            jax.ShapeDtypeStruct((_NU, _D), jnp.float32),
            jax.ShapeDtypeStruct((_NU, HH), jnp.float32),
            jax.ShapeDtypeStruct((_NU, HH), jnp.float32),
        ],
    )(x0, x1, x2, W1a, W1b, W1c, b1.reshape(1, _D), W2, b2.reshape(1, _D),
      Wc, bc.reshape(1, HH), Wt, bt.reshape(1, HH), WT_, bT_.reshape(1, HH),
      Woc, boc.reshape(1, _D), Wot, bot.reshape(1, _D), WoT, boT.reshape(1, _D))
    return outs


def kernel(xu, xp, edge_index, Wu, bu, Wp, bp, Wl0, bl0, Wr0, Wl1, bl1, Wr1,
           Wc1, bc1, Wc2, bc2, Wctl, bctl, Wtrt, btrt, WT, bT, Woc, boc,
           Wot, bot, WoT, boT):
    f32 = jnp.float32
    npad = _EPAD - _E
    srcp = jnp.pad(edge_index[0], (0, npad), constant_values=_N)
    dstp = jnp.pad(edge_index[1], (0, npad), constant_values=_N)
    pidx2d = jnp.bitwise_or(
        srcp, jnp.left_shift(dstp, 16)).reshape(_EPAD // _CHUNK, _CHUNK)

    xu_e = _tc_embed(xu, Wu, bu, 1000)
    xp_e = _tc_embed(xp, Wp, bp, 1000)
    emb0 = jnp.concatenate(
        [xu_e, xp_e, jnp.zeros((_NPADT - _N, _D), f32)], axis=0)

    agg0, cntp = _seg_sum_cnt(emb0, pidx2d)
    agg0 = agg0.reshape(_NCORES, _N, _D)
    arr = cntp.reshape(_NCORES, _NSUB, 640)
    cnt = jnp.concatenate(
        [arr[:, :_NSUB - 1, :_CPT].reshape(_NCORES, -1), arr[:, _NSUB - 1]],
        axis=1).T
    emb1 = _tc_sage(agg0, cnt, emb0[:_N], Wl0, bl0, Wr0)
    emb1p = jnp.concatenate([emb1, jnp.zeros((_NPADT - _N, _D), f32)], axis=0)
    agg1 = _seg_sum(emb1p, pidx2d)
    agg1 = agg1.reshape(_NCORES, _N, _D)
    emb2 = _tc_sage(agg1, cnt, emb1, Wl1, bl1, Wr1)

    pad_w = lambda w: jnp.pad(w, ((0, 0), (0, _D - w.shape[1])))
    pad_b = lambda b: jnp.pad(b, (0, _D - b.shape[0]))

    o_t1p, o_t0p, o_Tp, h_t1, h_t0 = _tc_head(
        xu_e, emb1[:_NU], emb2[:_NU],
        Wc1[0:_D], Wc1[_D:2 * _D], Wc1[2 * _D:3 * _D], bc1, Wc2, bc2,
        Wctl, bctl, Wtrt, btrt, WT, bT,
        pad_w(Woc), pad_b(boc), pad_w(Wot), pad_b(bot), pad_w(WoT), pad_b(boT))

    return (o_t1p[:, :1], o_t0p[:, :1], o_Tp[:, :1], h_t1, h_t0)


# speedup vs baseline: 2.2673x; 2.2673x over previous
import functools

import jax
import jax.numpy as jnp
from jax import lax
from jax.experimental import pallas as pl
from jax.experimental.pallas import tpu as pltpu
from jax.experimental.pallas import tpu_sc as plsc

_NU = 8000
_NP = 2000
_N = _NU + _NP
_E = 320000
_D = 128

_NCORES = 2
_NSUB = 16
_NTILES = _NCORES * _NSUB
_CHUNK = 80
_EPT = _E // _NTILES
_NCHUNK = _EPT // _CHUNK
_RPT = _N // _NSUB


_CPT = 624
_NPADC = _NSUB * 640


@functools.lru_cache(maxsize=None)
def _make_seg_sum(with_cnt):
    mesh = plsc.VectorSubcoreMesh(core_axis_name="c", subcore_axis_name="s",
                                  num_cores=_NCORES, num_subcores=_NSUB)
    out_type = [jax.ShapeDtypeStruct((_NCORES, _NSUB, _RPT, _D), jnp.float32)]
    scratch = [
        pltpu.VMEM((_CHUNK, _D), jnp.float32),
        pltpu.VMEM((_CHUNK, _D), jnp.float32),
        pltpu.VMEM((_CHUNK,), jnp.int32),
        pltpu.VMEM((_CHUNK,), jnp.int32),
        pltpu.VMEM((_CHUNK,), jnp.int32),
        pltpu.VMEM((_CHUNK,), jnp.int32),
        pltpu.VMEM_SHARED((_N, _D), jnp.float32),
        pltpu.SemaphoreType.DMA,
        pltpu.SemaphoreType.DMA,
    ]
    if with_cnt:
        out_type.append(
            jax.ShapeDtypeStruct((_NCORES, 1, _NPADC), jnp.float32))
        scratch += [
            pltpu.VMEM((_CHUNK,), jnp.float32),
            pltpu.VMEM((640,), jnp.float32),
            pltpu.VMEM_SHARED((_NPADC,), jnp.float32),
        ]

    def body(table, src, dst, zeros_a, agg_out, *rest):
        if with_cnt:
            cnt_out, gbuf0, gbuf1, sidx0, didx0, sidx1, didx1, acc, \
                sem0, sem1, onesv, zc, cacc = rest
        else:
            gbuf0, gbuf1, sidx0, didx0, sidx1, didx1, acc, sem0, sem1 = rest
            cnt_out = cacc = onesv = zc = None
        cid = lax.axis_index("c")
        sid = lax.axis_index("s")
        wid = cid * _NSUB + sid
        r0 = sid * _RPT
        base = wid * _EPT

        pltpu.sync_copy(zeros_a, acc.at[pl.ds(r0, _RPT)])
        if with_cnt:
            ones16 = jnp.full((16,), 1.0, jnp.float32)
            zeros16 = jnp.zeros((16,), jnp.float32)

            def fill_ones(i, c):
                onesv[pl.ds(i * 16, 16)] = ones16
                return c

            lax.fori_loop(0, _CHUNK // 16, fill_ones, 0)

            def fill_zero(i, c):
                zc[pl.ds(i * 16, 16)] = zeros16
                return c

            lax.fori_loop(0, 40, fill_zero, 0)
            pltpu.sync_copy(zc, cacc.at[pl.ds(sid * _CPT, 640)])
        plsc.subcore_barrier()

        bufs = ((gbuf0, sem0, sidx0, didx0), (gbuf1, sem1, sidx1, didx1))

        def load_and_fire(c, buf, sem, sidx, didx):
            off = base + c * _CHUNK
            pltpu.sync_copy(src.at[pl.ds(off, _CHUNK)], sidx)
            pltpu.sync_copy(dst.at[pl.ds(off, _CHUNK)], didx)
            pltpu.async_copy(table.at[sidx], buf, sem)

        def finish(c, buf, sem, sidx, didx, nxt):
            pltpu.make_async_copy(table.at[sidx], buf, sem).wait()
            pltpu.sync_copy(buf, acc.at[didx], add=True)
            if with_cnt:
                pltpu.sync_copy(onesv, cacc.at[didx], add=True)
            if nxt is not None:
                load_and_fire(nxt, buf, sem, sidx, didx)

        load_and_fire(0, *bufs[0])
        load_and_fire(1, *bufs[1])

        def group(g, carry):
            c = 2 * g
            finish(c, *bufs[0], nxt=c + 2)
            finish(c + 1, *bufs[1], nxt=c + 3)
            return carry

        lax.fori_loop(0, (_NCHUNK - 3) // 2, group, 0)
        finish(_NCHUNK - 3, *bufs[0], nxt=_NCHUNK - 1)
        finish(_NCHUNK - 2, *bufs[1], nxt=None)
        finish(_NCHUNK - 1, *bufs[0], nxt=None)
        plsc.subcore_barrier()

        pltpu.sync_copy(acc.at[pl.ds(r0, _RPT)], agg_out.at[cid, sid])
        if with_cnt:
            pltpu.sync_copy(cacc.at[pl.ds(sid * _CPT, 640)], zc)
            pltpu.sync_copy(zc, cnt_out.at[cid, 0, pl.ds(sid * 640, 640)])

    return pl.kernel(body, out_type, mesh=mesh, scratch_types=scratch)


def _seg_sum_cnt(*args):
    return _make_seg_sum(True)(*args)


def _seg_sum(*args):
    res = _make_seg_sum(False)(*args)
    return res[0] if isinstance(res, (list, tuple)) else res


def _tc_embed(x, W, b, block_rows):
    M, K = x.shape
    H = W.shape[1]

    def body(x_ref, w_ref, b_ref, o_ref):
        o_ref[...] = (jnp.dot(x_ref[...], w_ref[...],
                              preferred_element_type=jnp.float32) + b_ref[...])

    return pl.pallas_call(
        body,
        grid=(M // block_rows,),
        in_specs=[
            pl.BlockSpec((block_rows, K), lambda i: (i, 0)),
            pl.BlockSpec((K, H), lambda i: (0, 0)),
            pl.BlockSpec((1, H), lambda i: (0, 0)),
        ],
        out_specs=pl.BlockSpec((block_rows, H), lambda i: (i, 0)),
        out_shape=jax.ShapeDtypeStruct((M, H), jnp.float32),
    )(x, W, b.reshape(1, H))


def _tc_sage(agg, cnt, x, Wl, bl, Wr):
    BR = 1000

    def body(a_ref, c_ref, x_ref, wl_ref, bl_ref, wr_ref, o_ref):
        a = a_ref[0] + a_ref[1]
        c = c_ref[:, 0:1] + c_ref[:, 1:2]
        mean = a / jnp.maximum(c, 1.0)
        o_ref[...] = jnp.maximum(
            jnp.dot(mean, wl_ref[...], preferred_element_type=jnp.float32)
            + bl_ref[...]
            + jnp.dot(x_ref[...], wr_ref[...],
                      preferred_element_type=jnp.float32),
            0.0)

    return pl.pallas_call(
        body,
        grid=(_N // BR,),
        in_specs=[
            pl.BlockSpec((_NCORES, BR, _D), lambda i: (0, i, 0)),
            pl.BlockSpec((BR, _NCORES), lambda i: (i, 0)),
            pl.BlockSpec((BR, _D), lambda i: (i, 0)),
            pl.BlockSpec((_D, _D), lambda i: (0, 0)),
            pl.BlockSpec((1, _D), lambda i: (0, 0)),
            pl.BlockSpec((_D, _D), lambda i: (0, 0)),
        ],
        out_specs=pl.BlockSpec((BR, _D), lambda i: (i, 0)),
        out_shape=jax.ShapeDtypeStruct((_N, _D), jnp.float32),
    )(agg, cnt, x, Wl, bl.reshape(1, _D), Wr)


def _tc_head(x0, x1, x2, W1a, W1b, W1c, b1, W2, b2, Wc, bc, Wt, bt, WT_, bT_,
             Woc, boc, Wot, bot, WoT, boT):
    BR = 1000
    HH = 64

    def body(x0r, x1r, x2r, w1ar, w1br, w1cr, b1r, w2r, b2r, wcr, bcr,
             wtr, btr, wTr, bTr, wocr, bocr, wotr, botr, wTor, bTor,
             ot1, ot0, oT, ht1, ht0):
        dot = lambda a, w: jnp.dot(a, w, preferred_element_type=jnp.float32)
        h = jnp.maximum(dot(x0r[...], w1ar[...]) + dot(x1r[...], w1br[...])
                        + dot(x2r[...], w1cr[...]) + b1r[...], 0.0)
        h = jnp.maximum(dot(h, w2r[...]) + b2r[...], 0.0)
        a_t0 = jnp.maximum(dot(h, wcr[...]) + bcr[...], 0.0)
        a_t1 = jnp.maximum(dot(h, wtr[...]) + btr[...], 0.0)
        a_T = jnp.maximum(dot(h, wTr[...]) + bTr[...], 0.0)
        ht0[...] = a_t0
        ht1[...] = a_t1
        ot0[...] = jnp.maximum(dot(a_t0, wocr[...]) + bocr[...], 0.0)
        ot1[...] = jnp.maximum(dot(a_t1, wotr[...]) + botr[...], 0.0)
        oT[...] = jnp.maximum(dot(a_T, wTor[...]) + bTor[...], 0.0)

    full = lambda s: pl.BlockSpec(s, lambda i: tuple(0 for _ in s))
    row_spec = lambda w: pl.BlockSpec((BR, w), lambda i: (i, 0))
    outs = pl.pallas_call(
        body,
        grid=(_NU // BR,),
        in_specs=[
            row_spec(_D), row_spec(_D), row_spec(_D),
            full((_D, _D)), full((_D, _D)), full((_D, _D)), full((1, _D)),
            full((_D, _D)), full((1, _D)),
            full((_D, HH)), full((1, HH)),
            full((_D, HH)), full((1, HH)),
            full((_D, HH)), full((1, HH)),
            full((HH, _D)), full((1, _D)),
            full((HH, _D)), full((1, _D)),
            full((HH, _D)), full((1, _D)),
        ],
        out_specs=[
            row_spec(_D), row_spec(_D), row_spec(_D),
            row_spec(HH), row_spec(HH),
        ],
        out_shape=[
            jax.ShapeDtypeStruct((_NU, _D), jnp.float32),
            jax.ShapeDtypeStruct((_NU, _D), jnp.float32),
            jax.ShapeDtypeStruct((_NU, _D), jnp.float32),
            jax.ShapeDtypeStruct((_NU, HH), jnp.float32),
            jax.ShapeDtypeStruct((_NU, HH), jnp.float32),
        ],
    )(x0, x1, x2, W1a, W1b, W1c, b1.reshape(1, _D), W2, b2.reshape(1, _D),
      Wc, bc.reshape(1, HH), Wt, bt.reshape(1, HH), WT_, bT_.reshape(1, HH),
      Woc, boc.reshape(1, _D), Wot, bot.reshape(1, _D), WoT, boT.reshape(1, _D))
    return outs


def kernel(xu, xp, edge_index, Wu, bu, Wp, bp, Wl0, bl0, Wr0, Wl1, bl1, Wr1,
           Wc1, bc1, Wc2, bc2, Wctl, bctl, Wtrt, btrt, WT, bT, Woc, boc,
           Wot, bot, WoT, boT):
    f32 = jnp.float32
    src = edge_index[0]
    dst = edge_index[1]

    xu_e = _tc_embed(xu, Wu, bu, 1000)
    xp_e = _tc_embed(xp, Wp, bp, 1000)
    emb0 = jnp.concatenate([xu_e, xp_e], axis=0)
    zeros_a = jnp.zeros((_RPT, _D), f32)

    agg0, cntp = _seg_sum_cnt(emb0, src, dst, zeros_a)
    agg0 = agg0.reshape(_NCORES, _N, _D)
    arr = cntp.reshape(_NCORES, _NSUB, 640)
    cnt = jnp.concatenate(
        [arr[:, :_NSUB - 1, :_CPT].reshape(_NCORES, -1), arr[:, _NSUB - 1]],
        axis=1).T
    emb1 = _tc_sage(agg0, cnt, emb0, Wl0, bl0, Wr0)
    agg1 = _seg_sum(emb1, src, dst, zeros_a)
    agg1 = agg1.reshape(_NCORES, _N, _D)
    emb2 = _tc_sage(agg1, cnt, emb1, Wl1, bl1, Wr1)

    pad_w = lambda w: jnp.pad(w, ((0, 0), (0, _D - w.shape[1])))
    pad_b = lambda b: jnp.pad(b, (0, _D - b.shape[0]))

    o_t1p, o_t0p, o_Tp, h_t1, h_t0 = _tc_head(
        xu_e, emb1[:_NU], emb2[:_NU],
        Wc1[0:_D], Wc1[_D:2 * _D], Wc1[2 * _D:3 * _D], bc1, Wc2, bc2,
        Wctl, bctl, Wtrt, btrt, WT, bT,
        pad_w(Woc), pad_b(boc), pad_w(Wot), pad_b(bot), pad_w(WoT), pad_b(boT))

    return (o_t1p[:, :1], o_t0p[:, :1], o_Tp[:, :1], h_t1, h_t0)


# speedup vs baseline: 2.2773x; 1.0044x over previous
import functools

import jax
import jax.numpy as jnp
from jax import lax
from jax.experimental import pallas as pl
from jax.experimental.pallas import tpu as pltpu
from jax.experimental.pallas import tpu_sc as plsc

_NU = 8000
_NP = 2000
_N = _NU + _NP
_E = 320000
_D = 128

_NCORES = 2
_NSUB = 16
_NTILES = _NCORES * _NSUB
_CHUNK = 80
_EPT = _E // _NTILES
_NCHUNK = _EPT // _CHUNK
_RPT = _N // _NSUB


_CPT = 624
_NPADC = _NSUB * 640


@functools.lru_cache(maxsize=None)
def _make_seg_sum(with_cnt):
    mesh = plsc.VectorSubcoreMesh(core_axis_name="c", subcore_axis_name="s",
                                  num_cores=_NCORES, num_subcores=_NSUB)
    out_type = [jax.ShapeDtypeStruct((_NCORES, _NSUB, _RPT, _D), jnp.float32)]
    scratch = [
        pltpu.VMEM((_CHUNK, _D), jnp.float32),
        pltpu.VMEM((_CHUNK, _D), jnp.float32),
        pltpu.VMEM((_CHUNK, _D), jnp.float32),
        pltpu.VMEM((_CHUNK,), jnp.int32),
        pltpu.VMEM((_CHUNK,), jnp.int32),
        pltpu.VMEM((_CHUNK,), jnp.int32),
        pltpu.VMEM((_CHUNK,), jnp.int32),
        pltpu.VMEM((_CHUNK,), jnp.int32),
        pltpu.VMEM((_CHUNK,), jnp.int32),
        pltpu.VMEM_SHARED((_N, _D), jnp.float32),
        pltpu.SemaphoreType.DMA,
        pltpu.SemaphoreType.DMA,
        pltpu.SemaphoreType.DMA,
    ]
    if with_cnt:
        out_type.append(
            jax.ShapeDtypeStruct((_NCORES, 1, _NPADC), jnp.float32))
        scratch += [
            pltpu.VMEM((_CHUNK,), jnp.float32),
            pltpu.VMEM((640,), jnp.float32),
            pltpu.VMEM_SHARED((_NPADC,), jnp.float32),
        ]

    def body(table, src, dst, zeros_a, agg_out, *rest):
        if with_cnt:
            cnt_out, gbuf0, gbuf1, gbuf2, sidx0, didx0, sidx1, didx1, sidx2, \
                didx2, acc, sem0, sem1, sem2, onesv, zc, cacc = rest
        else:
            gbuf0, gbuf1, gbuf2, sidx0, didx0, sidx1, didx1, sidx2, didx2, \
                acc, sem0, sem1, sem2 = rest
            cnt_out = cacc = onesv = zc = None
        cid = lax.axis_index("c")
        sid = lax.axis_index("s")
        wid = cid * _NSUB + sid
        r0 = sid * _RPT
        base = wid * _EPT

        pltpu.sync_copy(zeros_a, acc.at[pl.ds(r0, _RPT)])
        if with_cnt:
            ones16 = jnp.full((16,), 1.0, jnp.float32)
            zeros16 = jnp.zeros((16,), jnp.float32)

            def fill_ones(i, c):
                onesv[pl.ds(i * 16, 16)] = ones16
                return c

            lax.fori_loop(0, _CHUNK // 16, fill_ones, 0)

            def fill_zero(i, c):
                zc[pl.ds(i * 16, 16)] = zeros16
                return c

            lax.fori_loop(0, 40, fill_zero, 0)
            pltpu.sync_copy(zc, cacc.at[pl.ds(sid * _CPT, 640)])
        plsc.subcore_barrier()

        bufs = ((gbuf0, sem0, sidx0, didx0), (gbuf1, sem1, sidx1, didx1),
                (gbuf2, sem2, sidx2, didx2))

        def load_and_fire(c, buf, sem, sidx, didx):
            off = base + c * _CHUNK
            pltpu.sync_copy(src.at[pl.ds(off, _CHUNK)], sidx)
            pltpu.sync_copy(dst.at[pl.ds(off, _CHUNK)], didx)
            pltpu.async_copy(table.at[sidx], buf, sem)

        def finish(c, buf, sem, sidx, didx, nxt):
            pltpu.make_async_copy(table.at[sidx], buf, sem).wait()
            pltpu.sync_copy(buf, acc.at[didx], add=True)
            if with_cnt:
                pltpu.sync_copy(onesv, cacc.at[didx], add=True)
            if nxt is not None:
                load_and_fire(nxt, buf, sem, sidx, didx)

        for b in range(3):
            load_and_fire(b, *bufs[b])

        def group(g, carry):
            c = 3 * g
            for b in range(3):
                finish(c + b, *bufs[b], nxt=c + b + 3)
            return carry

        lax.fori_loop(0, (_NCHUNK - 5) // 3, group, 0)
        finish(_NCHUNK - 5, *bufs[0], nxt=_NCHUNK - 2)
        finish(_NCHUNK - 4, *bufs[1], nxt=_NCHUNK - 1)
        finish(_NCHUNK - 3, *bufs[2], nxt=None)
        finish(_NCHUNK - 2, *bufs[0], nxt=None)
        finish(_NCHUNK - 1, *bufs[1], nxt=None)
        plsc.subcore_barrier()

        pltpu.sync_copy(acc.at[pl.ds(r0, _RPT)], agg_out.at[cid, sid])
        if with_cnt:
            pltpu.sync_copy(cacc.at[pl.ds(sid * _CPT, 640)], zc)
            pltpu.sync_copy(zc, cnt_out.at[cid, 0, pl.ds(sid * 640, 640)])

    return pl.kernel(body, out_type, mesh=mesh, scratch_types=scratch)


def _seg_sum_cnt(*args):
    return _make_seg_sum(True)(*args)


def _seg_sum(*args):
    res = _make_seg_sum(False)(*args)
    return res[0] if isinstance(res, (list, tuple)) else res


def _tc_embed(x, W, b, block_rows):
    M, K = x.shape
    H = W.shape[1]

    def body(x_ref, w_ref, b_ref, o_ref):
        o_ref[...] = (jnp.dot(x_ref[...], w_ref[...],
                              preferred_element_type=jnp.float32) + b_ref[...])

    return pl.pallas_call(
        body,
        grid=(M // block_rows,),
        in_specs=[
            pl.BlockSpec((block_rows, K), lambda i: (i, 0)),
            pl.BlockSpec((K, H), lambda i: (0, 0)),
            pl.BlockSpec((1, H), lambda i: (0, 0)),
        ],
        out_specs=pl.BlockSpec((block_rows, H), lambda i: (i, 0)),
        out_shape=jax.ShapeDtypeStruct((M, H), jnp.float32),
    )(x, W, b.reshape(1, H))


def _tc_sage(agg, cnt, x, Wl, bl, Wr):
    BR = 1000

    def body(a_ref, c_ref, x_ref, wl_ref, bl_ref, wr_ref, o_ref):
        a = a_ref[0] + a_ref[1]
        c = c_ref[:, 0:1] + c_ref[:, 1:2]
        mean = a / jnp.maximum(c, 1.0)
        o_ref[...] = jnp.maximum(
            jnp.dot(mean, wl_ref[...], preferred_element_type=jnp.float32)
            + bl_ref[...]
            + jnp.dot(x_ref[...], wr_ref[...],
                      preferred_element_type=jnp.float32),
            0.0)

    return pl.pallas_call(
        body,
        grid=(_N // BR,),
        in_specs=[
            pl.BlockSpec((_NCORES, BR, _D), lambda i: (0, i, 0)),
            pl.BlockSpec((BR, _NCORES), lambda i: (i, 0)),
            pl.BlockSpec((BR, _D), lambda i: (i, 0)),
            pl.BlockSpec((_D, _D), lambda i: (0, 0)),
            pl.BlockSpec((1, _D), lambda i: (0, 0)),
            pl.BlockSpec((_D, _D), lambda i: (0, 0)),
        ],
        out_specs=pl.BlockSpec((BR, _D), lambda i: (i, 0)),
        out_shape=jax.ShapeDtypeStruct((_N, _D), jnp.float32),
    )(agg, cnt, x, Wl, bl.reshape(1, _D), Wr)


def _tc_head(x0, x1, x2, W1a, W1b, W1c, b1, W2, b2, Wc, bc, Wt, bt, WT_, bT_,
             Woc, boc, Wot, bot, WoT, boT):
    BR = 1000
    HH = 64

    def body(x0r, x1r, x2r, w1ar, w1br, w1cr, b1r, w2r, b2r, wcr, bcr,
             wtr, btr, wTr, bTr, wocr, bocr, wotr, botr, wTor, bTor,
             ot1, ot0, oT, ht1, ht0):
        dot = lambda a, w: jnp.dot(a, w, preferred_element_type=jnp.float32)
        h = jnp.maximum(dot(x0r[...], w1ar[...]) + dot(x1r[...], w1br[...])
                        + dot(x2r[...], w1cr[...]) + b1r[...], 0.0)
        h = jnp.maximum(dot(h, w2r[...]) + b2r[...], 0.0)
        a_t0 = jnp.maximum(dot(h, wcr[...]) + bcr[...], 0.0)
        a_t1 = jnp.maximum(dot(h, wtr[...]) + btr[...], 0.0)
        a_T = jnp.maximum(dot(h, wTr[...]) + bTr[...], 0.0)
        ht0[...] = a_t0
        ht1[...] = a_t1
        ot0[...] = jnp.maximum(dot(a_t0, wocr[...]) + bocr[...], 0.0)
        ot1[...] = jnp.maximum(dot(a_t1, wotr[...]) + botr[...], 0.0)
        oT[...] = jnp.maximum(dot(a_T, wTor[...]) + bTor[...], 0.0)

    full = lambda s: pl.BlockSpec(s, lambda i: tuple(0 for _ in s))
    row_spec = lambda w: pl.BlockSpec((BR, w), lambda i: (i, 0))
    outs = pl.pallas_call(
        body,
        grid=(_NU // BR,),
        in_specs=[
            row_spec(_D), row_spec(_D), row_spec(_D),
            full((_D, _D)), full((_D, _D)), full((_D, _D)), full((1, _D)),
            full((_D, _D)), full((1, _D)),
            full((_D, HH)), full((1, HH)),
            full((_D, HH)), full((1, HH)),
            full((_D, HH)), full((1, HH)),
            full((HH, _D)), full((1, _D)),
            full((HH, _D)), full((1, _D)),
            full((HH, _D)), full((1, _D)),
        ],
        out_specs=[
            row_spec(_D), row_spec(_D), row_spec(_D),
            row_spec(HH), row_spec(HH),
        ],
        out_shape=[
            jax.ShapeDtypeStruct((_NU, _D), jnp.float32),
            jax.ShapeDtypeStruct((_NU, _D), jnp.float32),
            jax.ShapeDtypeStruct((_NU, _D), jnp.float32),
            jax.ShapeDtypeStruct((_NU, HH), jnp.float32),
            jax.ShapeDtypeStruct((_NU, HH), jnp.float32),
        ],
    )(x0, x1, x2, W1a, W1b, W1c, b1.reshape(1, _D), W2, b2.reshape(1, _D),
      Wc, bc.reshape(1, HH), Wt, bt.reshape(1, HH), WT_, bT_.reshape(1, HH),
      Woc, boc.reshape(1, _D), Wot, bot.reshape(1, _D), WoT, boT.reshape(1, _D))
    return outs


def kernel(xu, xp, edge_index, Wu, bu, Wp, bp, Wl0, bl0, Wr0, Wl1, bl1, Wr1,
           Wc1, bc1, Wc2, bc2, Wctl, bctl, Wtrt, btrt, WT, bT, Woc, boc,
           Wot, bot, WoT, boT):
    f32 = jnp.float32
    src = edge_index[0]
    dst = edge_index[1]

    xu_e = _tc_embed(xu, Wu, bu, 1000)
    xp_e = _tc_embed(xp, Wp, bp, 1000)
    emb0 = jnp.concatenate([xu_e, xp_e], axis=0)
    zeros_a = jnp.zeros((_RPT, _D), f32)

    agg0, cntp = _seg_sum_cnt(emb0, src, dst, zeros_a)
    agg0 = agg0.reshape(_NCORES, _N, _D)
    arr = cntp.reshape(_NCORES, _NSUB, 640)
    cnt = jnp.concatenate(
        [arr[:, :_NSUB - 1, :_CPT].reshape(_NCORES, -1), arr[:, _NSUB - 1]],
        axis=1).T
    emb1 = _tc_sage(agg0, cnt, emb0, Wl0, bl0, Wr0)
    agg1 = _seg_sum(emb1, src, dst, zeros_a)
    agg1 = agg1.reshape(_NCORES, _N, _D)
    emb2 = _tc_sage(agg1, cnt, emb1, Wl1, bl1, Wr1)

    pad_w = lambda w: jnp.pad(w, ((0, 0), (0, _D - w.shape[1])))
    pad_b = lambda b: jnp.pad(b, (0, _D - b.shape[0]))

    o_t1p, o_t0p, o_Tp, h_t1, h_t0 = _tc_head(
        xu_e, emb1[:_NU], emb2[:_NU],
        Wc1[0:_D], Wc1[_D:2 * _D], Wc1[2 * _D:3 * _D], bc1, Wc2, bc2,
        Wctl, bctl, Wtrt, btrt, WT, bT,
        pad_w(Woc), pad_b(boc), pad_w(Wot), pad_b(bot), pad_w(WoT), pad_b(boT))

    return (o_t1p[:, :1], o_t0p[:, :1], o_Tp[:, :1], h_t1, h_t0)


# speedup vs baseline: 2.6783x; 1.1761x over previous
import functools

import jax
import jax.numpy as jnp
from jax import lax
from jax.experimental import pallas as pl
from jax.experimental.pallas import tpu as pltpu
from jax.experimental.pallas import tpu_sc as plsc

_NU = 8000
_NP = 2000
_N = _NU + _NP
_E = 320000
_D = 128

_NCORES = 2
_NSUB = 16
_NTILES = _NCORES * _NSUB
_CHUNK = 80
_EPT = _E // _NTILES
_NCHUNK = _EPT // _CHUNK
_RPT = _N // _NSUB


_CPT = 624
_NPADC = _NSUB * 640


@functools.lru_cache(maxsize=None)
def _make_seg_sum(with_cnt):
    mesh = plsc.VectorSubcoreMesh(core_axis_name="c", subcore_axis_name="s",
                                  num_cores=_NCORES, num_subcores=_NSUB)
    out_type = [jax.ShapeDtypeStruct((_NCORES, _NSUB, _RPT, _D), jnp.float32)]
    scratch = [
        pltpu.VMEM((_CHUNK, _D), jnp.float32),
        pltpu.VMEM((_CHUNK, _D), jnp.float32),
        pltpu.VMEM((_CHUNK, _D), jnp.float32),
        pltpu.VMEM((_CHUNK,), jnp.int32),
        pltpu.VMEM((_CHUNK,), jnp.int32),
        pltpu.VMEM((_CHUNK,), jnp.int32),
        pltpu.VMEM((_CHUNK,), jnp.int32),
        pltpu.VMEM((_CHUNK,), jnp.int32),
        pltpu.VMEM((_CHUNK,), jnp.int32),
        pltpu.VMEM_SHARED((_N, _D), jnp.float32),
        pltpu.SemaphoreType.DMA,
        pltpu.SemaphoreType.DMA,
        pltpu.SemaphoreType.DMA,
        pltpu.SemaphoreType.DMA,
        pltpu.SemaphoreType.DMA,
        pltpu.SemaphoreType.DMA,
    ]
    if with_cnt:
        out_type.append(
            jax.ShapeDtypeStruct((_NCORES, 1, _NPADC), jnp.float32))
        scratch += [
            pltpu.VMEM((_CHUNK,), jnp.float32),
            pltpu.VMEM((640,), jnp.float32),
            pltpu.VMEM_SHARED((_NPADC,), jnp.float32),
        ]

    def body(table, src, dst, zeros_a, agg_out, *rest):
        if with_cnt:
            cnt_out, gbuf0, gbuf1, gbuf2, sidx0, didx0, sidx1, didx1, sidx2, \
                didx2, acc, sem0, sem1, sem2, ssm0, ssm1, ssm2, onesv, zc, \
                cacc = rest
        else:
            gbuf0, gbuf1, gbuf2, sidx0, didx0, sidx1, didx1, sidx2, didx2, \
                acc, sem0, sem1, sem2, ssm0, ssm1, ssm2 = rest
            cnt_out = cacc = onesv = zc = None
        cid = lax.axis_index("c")
        sid = lax.axis_index("s")
        wid = cid * _NSUB + sid
        r0 = sid * _RPT
        base = wid * _EPT

        pltpu.sync_copy(zeros_a, acc.at[pl.ds(r0, _RPT)])
        if with_cnt:
            ones16 = jnp.full((16,), 1.0, jnp.float32)
            zeros16 = jnp.zeros((16,), jnp.float32)

            def fill_ones(i, c):
                onesv[pl.ds(i * 16, 16)] = ones16
                return c

            lax.fori_loop(0, _CHUNK // 16, fill_ones, 0)

            def fill_zero(i, c):
                zc[pl.ds(i * 16, 16)] = zeros16
                return c

            lax.fori_loop(0, 40, fill_zero, 0)
            pltpu.sync_copy(zc, cacc.at[pl.ds(sid * _CPT, 640)])
        plsc.subcore_barrier()

        bufs = ((gbuf0, sem0, ssm0, sidx0, didx0),
                (gbuf1, sem1, ssm1, sidx1, didx1),
                (gbuf2, sem2, ssm2, sidx2, didx2))

        def load_and_fire(c, buf, sem, ssm, sidx, didx):
            off = base + c * _CHUNK
            pltpu.sync_copy(src.at[pl.ds(off, _CHUNK)], sidx)
            pltpu.sync_copy(dst.at[pl.ds(off, _CHUNK)], didx)
            pltpu.async_copy(table.at[sidx], buf, sem)

        def wait_scatters(buf, ssm, didx):
            pltpu.make_async_copy(buf, acc.at[didx], ssm).wait()
            if with_cnt:
                pltpu.make_async_copy(onesv, cacc.at[didx], ssm).wait()

        def visit(c, B, Bn, wait_sc, fire_next):
            buf, sem, ssm, sidx, didx = B
            if fire_next:
                if wait_sc:
                    wait_scatters(Bn[0], Bn[2], Bn[4])
                load_and_fire(c + 1, *Bn)
            pltpu.make_async_copy(table.at[sidx], buf, sem).wait()
            pltpu.async_copy(buf, acc.at[didx], ssm, add=True)
            if with_cnt:
                pltpu.async_copy(onesv, cacc.at[didx], ssm, add=True)

        load_and_fire(0, *bufs[0])
        visit(0, bufs[0], bufs[1], False, True)
        visit(1, bufs[1], bufs[2], False, True)

        def group(g, carry):
            c = 3 * g + 2
            for b in range(3):
                visit(c + b, bufs[(2 + b) % 3], bufs[(3 + b) % 3],
                      True, True)
            return carry

        lax.fori_loop(0, (_NCHUNK - 5) // 3, group, 0)
        visit(_NCHUNK - 3, bufs[(_NCHUNK - 3) % 3], bufs[(_NCHUNK - 2) % 3],
              True, True)
        visit(_NCHUNK - 2, bufs[(_NCHUNK - 2) % 3], bufs[(_NCHUNK - 1) % 3],
              True, True)
        visit(_NCHUNK - 1, bufs[(_NCHUNK - 1) % 3], None, False, False)
        for b in range(3):
            wait_scatters(bufs[b][0], bufs[b][2], bufs[b][4])
        plsc.subcore_barrier()

        pltpu.sync_copy(acc.at[pl.ds(r0, _RPT)], agg_out.at[cid, sid])
        if with_cnt:
            pltpu.sync_copy(cacc.at[pl.ds(sid * _CPT, 640)], zc)
            pltpu.sync_copy(zc, cnt_out.at[cid, 0, pl.ds(sid * 640, 640)])

    return pl.kernel(body, out_type, mesh=mesh, scratch_types=scratch)


def _seg_sum_cnt(*args):
    return _make_seg_sum(True)(*args)


def _seg_sum(*args):
    res = _make_seg_sum(False)(*args)
    return res[0] if isinstance(res, (list, tuple)) else res


def _tc_embed(x, W, b, block_rows):
    M, K = x.shape
    H = W.shape[1]

    def body(x_ref, w_ref, b_ref, o_ref):
        o_ref[...] = (jnp.dot(x_ref[...], w_ref[...],
                              preferred_element_type=jnp.float32) + b_ref[...])

    return pl.pallas_call(
        body,
        grid=(M // block_rows,),
        in_specs=[
            pl.BlockSpec((block_rows, K), lambda i: (i, 0)),
            pl.BlockSpec((K, H), lambda i: (0, 0)),
            pl.BlockSpec((1, H), lambda i: (0, 0)),
        ],
        out_specs=pl.BlockSpec((block_rows, H), lambda i: (i, 0)),
        out_shape=jax.ShapeDtypeStruct((M, H), jnp.float32),
    )(x, W, b.reshape(1, H))


def _tc_sage(agg, cnt, x, Wl, bl, Wr):
    BR = 1000

    def body(a_ref, c_ref, x_ref, wl_ref, bl_ref, wr_ref, o_ref):
        a = a_ref[0] + a_ref[1]
        c = c_ref[:, 0:1] + c_ref[:, 1:2]
        mean = a / jnp.maximum(c, 1.0)
        o_ref[...] = jnp.maximum(
            jnp.dot(mean, wl_ref[...], preferred_element_type=jnp.float32)
            + bl_ref[...]
            + jnp.dot(x_ref[...], wr_ref[...],
                      preferred_element_type=jnp.float32),
            0.0)

    return pl.pallas_call(
        body,
        grid=(_N // BR,),
        in_specs=[
            pl.BlockSpec((_NCORES, BR, _D), lambda i: (0, i, 0)),
            pl.BlockSpec((BR, _NCORES), lambda i: (i, 0)),
            pl.BlockSpec((BR, _D), lambda i: (i, 0)),
            pl.BlockSpec((_D, _D), lambda i: (0, 0)),
            pl.BlockSpec((1, _D), lambda i: (0, 0)),
            pl.BlockSpec((_D, _D), lambda i: (0, 0)),
        ],
        out_specs=pl.BlockSpec((BR, _D), lambda i: (i, 0)),
        out_shape=jax.ShapeDtypeStruct((_N, _D), jnp.float32),
    )(agg, cnt, x, Wl, bl.reshape(1, _D), Wr)


def _tc_head(x0, x1, x2, W1a, W1b, W1c, b1, W2, b2, Wc, bc, Wt, bt, WT_, bT_,
             Woc, boc, Wot, bot, WoT, boT):
    BR = 1000
    HH = 64

    def body(x0r, x1r, x2r, w1ar, w1br, w1cr, b1r, w2r, b2r, wcr, bcr,
             wtr, btr, wTr, bTr, wocr, bocr, wotr, botr, wTor, bTor,
             ot1, ot0, oT, ht1, ht0):
        dot = lambda a, w: jnp.dot(a, w, preferred_element_type=jnp.float32)
        h = jnp.maximum(dot(x0r[...], w1ar[...]) + dot(x1r[...], w1br[...])
                        + dot(x2r[...], w1cr[...]) + b1r[...], 0.0)
        h = jnp.maximum(dot(h, w2r[...]) + b2r[...], 0.0)
        a_t0 = jnp.maximum(dot(h, wcr[...]) + bcr[...], 0.0)
        a_t1 = jnp.maximum(dot(h, wtr[...]) + btr[...], 0.0)
        a_T = jnp.maximum(dot(h, wTr[...]) + bTr[...], 0.0)
        ht0[...] = a_t0
        ht1[...] = a_t1
        ot0[...] = jnp.maximum(dot(a_t0, wocr[...]) + bocr[...], 0.0)
        ot1[...] = jnp.maximum(dot(a_t1, wotr[...]) + botr[...], 0.0)
        oT[...] = jnp.maximum(dot(a_T, wTor[...]) + bTor[...], 0.0)

    full = lambda s: pl.BlockSpec(s, lambda i: tuple(0 for _ in s))
    row_spec = lambda w: pl.BlockSpec((BR, w), lambda i: (i, 0))
    outs = pl.pallas_call(
        body,
        grid=(_NU // BR,),
        in_specs=[
            row_spec(_D), row_spec(_D), row_spec(_D),
            full((_D, _D)), full((_D, _D)), full((_D, _D)), full((1, _D)),
            full((_D, _D)), full((1, _D)),
            full((_D, HH)), full((1, HH)),
            full((_D, HH)), full((1, HH)),
            full((_D, HH)), full((1, HH)),
            full((HH, _D)), full((1, _D)),
            full((HH, _D)), full((1, _D)),
            full((HH, _D)), full((1, _D)),
        ],
        out_specs=[
            row_spec(_D), row_spec(_D), row_spec(_D),
            row_spec(HH), row_spec(HH),
        ],
        out_shape=[
            jax.ShapeDtypeStruct((_NU, _D), jnp.float32),
            jax.ShapeDtypeStruct((_NU, _D), jnp.float32),
            jax.ShapeDtypeStruct((_NU, _D), jnp.float32),
            jax.ShapeDtypeStruct((_NU, HH), jnp.float32),
            jax.ShapeDtypeStruct((_NU, HH), jnp.float32),
        ],
    )(x0, x1, x2, W1a, W1b, W1c, b1.reshape(1, _D), W2, b2.reshape(1, _D),
      Wc, bc.reshape(1, HH), Wt, bt.reshape(1, HH), WT_, bT_.reshape(1, HH),
      Woc, boc.reshape(1, _D), Wot, bot.reshape(1, _D), WoT, boT.reshape(1, _D))
    return outs


def kernel(xu, xp, edge_index, Wu, bu, Wp, bp, Wl0, bl0, Wr0, Wl1, bl1, Wr1,
           Wc1, bc1, Wc2, bc2, Wctl, bctl, Wtrt, btrt, WT, bT, Woc, boc,
           Wot, bot, WoT, boT):
    f32 = jnp.float32
    src = edge_index[0]
    dst = edge_index[1]

    xu_e = _tc_embed(xu, Wu, bu, 1000)
    xp_e = _tc_embed(xp, Wp, bp, 1000)
    emb0 = jnp.concatenate([xu_e, xp_e], axis=0)
    zeros_a = jnp.zeros((_RPT, _D), f32)

    agg0, cntp = _seg_sum_cnt(emb0, src, dst, zeros_a)
    agg0 = agg0.reshape(_NCORES, _N, _D)
    arr = cntp.reshape(_NCORES, _NSUB, 640)
    cnt = jnp.concatenate(
        [arr[:, :_NSUB - 1, :_CPT].reshape(_NCORES, -1), arr[:, _NSUB - 1]],
        axis=1).T
    emb1 = _tc_sage(agg0, cnt, emb0, Wl0, bl0, Wr0)
    agg1 = _seg_sum(emb1, src, dst, zeros_a)
    agg1 = agg1.reshape(_NCORES, _N, _D)
    emb2 = _tc_sage(agg1, cnt, emb1, Wl1, bl1, Wr1)

    pad_w = lambda w: jnp.pad(w, ((0, 0), (0, _D - w.shape[1])))
    pad_b = lambda b: jnp.pad(b, (0, _D - b.shape[0]))

    o_t1p, o_t0p, o_Tp, h_t1, h_t0 = _tc_head(
        xu_e, emb1[:_NU], emb2[:_NU],
        Wc1[0:_D], Wc1[_D:2 * _D], Wc1[2 * _D:3 * _D], bc1, Wc2, bc2,
        Wctl, bctl, Wtrt, btrt, WT, bT,
        pad_w(Woc), pad_b(boc), pad_w(Wot), pad_b(bot), pad_w(WoT), pad_b(boT))

    return (o_t1p[:, :1], o_t0p[:, :1], o_Tp[:, :1], h_t1, h_t0)


# speedup vs baseline: 2.7270x; 1.0182x over previous
import functools

import jax
import jax.numpy as jnp
from jax import lax
from jax.experimental import pallas as pl
from jax.experimental.pallas import tpu as pltpu
from jax.experimental.pallas import tpu_sc as plsc

_NU = 8000
_NP = 2000
_N = _NU + _NP
_E = 320000
_D = 128

_NCORES = 2
_NSUB = 16
_NTILES = _NCORES * _NSUB
_CHUNK = 80
_EPT = _E // _NTILES
_NCHUNK = _EPT // _CHUNK
_RPT = _N // _NSUB


_CPT = 624
_NPADC = _NSUB * 640


@functools.lru_cache(maxsize=None)
def _make_seg_sum(with_cnt):
    mesh = plsc.VectorSubcoreMesh(core_axis_name="c", subcore_axis_name="s",
                                  num_cores=_NCORES, num_subcores=_NSUB)
    out_type = [jax.ShapeDtypeStruct((_NCORES, _NSUB, _RPT, _D), jnp.float32)]
    scratch = [
        pltpu.VMEM((_CHUNK, _D), jnp.float32),
        pltpu.VMEM((_CHUNK, _D), jnp.float32),
        pltpu.VMEM((_CHUNK, _D), jnp.float32),
        pltpu.VMEM((_CHUNK,), jnp.int32),
        pltpu.VMEM((_CHUNK,), jnp.int32),
        pltpu.VMEM((_CHUNK,), jnp.int32),
        pltpu.VMEM((_CHUNK,), jnp.int32),
        pltpu.VMEM((_CHUNK,), jnp.int32),
        pltpu.VMEM((_CHUNK,), jnp.int32),
        pltpu.VMEM_SHARED((_N, _D), jnp.float32),
        pltpu.SemaphoreType.DMA,
        pltpu.SemaphoreType.DMA,
        pltpu.SemaphoreType.DMA,
        pltpu.SemaphoreType.DMA,
        pltpu.SemaphoreType.DMA,
        pltpu.SemaphoreType.DMA,
    ]
    if with_cnt:
        out_type.append(
            jax.ShapeDtypeStruct((_NCORES, 1, _NPADC), jnp.float32))
        scratch += [
            pltpu.VMEM((_CHUNK,), jnp.float32),
            pltpu.VMEM((640,), jnp.float32),
            pltpu.VMEM_SHARED((_NPADC,), jnp.float32),
        ]

    def body(table, src, dst, zeros_a, agg_out, *rest):
        if with_cnt:
            cnt_out, gbuf0, gbuf1, gbuf2, sidx0, didx0, sidx1, didx1, sidx2, \
                didx2, acc, sem0, sem1, sem2, ssm0, ssm1, ssm2, onesv, zc, \
                cacc = rest
        else:
            gbuf0, gbuf1, gbuf2, sidx0, didx0, sidx1, didx1, sidx2, didx2, \
                acc, sem0, sem1, sem2, ssm0, ssm1, ssm2 = rest
            cnt_out = cacc = onesv = zc = None
        cid = lax.axis_index("c")
        sid = lax.axis_index("s")
        wid = cid * _NSUB + sid
        r0 = sid * _RPT
        base = wid * _EPT

        pltpu.sync_copy(zeros_a, acc.at[pl.ds(r0, _RPT)])
        if with_cnt:
            ones16 = jnp.full((16,), 1.0, jnp.float32)
            zeros16 = jnp.zeros((16,), jnp.float32)

            def fill_ones(i, c):
                onesv[pl.ds(i * 16, 16)] = ones16
                return c

            lax.fori_loop(0, _CHUNK // 16, fill_ones, 0)

            def fill_zero(i, c):
                zc[pl.ds(i * 16, 16)] = zeros16
                return c

            lax.fori_loop(0, 40, fill_zero, 0)
            pltpu.sync_copy(zc, cacc.at[pl.ds(sid * _CPT, 640)])
        plsc.subcore_barrier()

        bufs = ((gbuf0, sem0, ssm0, sidx0, didx0),
                (gbuf1, sem1, ssm1, sidx1, didx1),
                (gbuf2, sem2, ssm2, sidx2, didx2))

        def load_and_fire(c, buf, sem, ssm, sidx, didx):
            off = base + c * _CHUNK
            pltpu.sync_copy(src.at[pl.ds(off, _CHUNK)], sidx)
            pltpu.sync_copy(dst.at[pl.ds(off, _CHUNK)], didx)
            pltpu.async_copy(table.at[sidx], buf, sem)

        def wait_scatters(buf, ssm, didx):
            pltpu.make_async_copy(buf, acc.at[didx], ssm).wait()
            if with_cnt:
                pltpu.make_async_copy(onesv, cacc.at[didx], ssm).wait()

        def visit(c, B, Bn, wait_sc, fire_next):
            buf, sem, ssm, sidx, didx = B
            if fire_next:
                if wait_sc:
                    wait_scatters(Bn[0], Bn[2], Bn[4])
                load_and_fire(c + 1, *Bn)
            pltpu.make_async_copy(table.at[sidx], buf, sem).wait()
            pltpu.async_copy(buf, acc.at[didx], ssm, add=True)
            if with_cnt:
                pltpu.async_copy(onesv, cacc.at[didx], ssm, add=True)

        load_and_fire(0, *bufs[0])
        visit(0, bufs[0], bufs[1], False, True)
        visit(1, bufs[1], bufs[2], False, True)

        def group(g, carry):
            c = 3 * g + 2
            for b in range(3):
                visit(c + b, bufs[(2 + b) % 3], bufs[(3 + b) % 3],
                      True, True)
            return carry

        lax.fori_loop(0, (_NCHUNK - 5) // 3, group, 0)
        visit(_NCHUNK - 3, bufs[(_NCHUNK - 3) % 3], bufs[(_NCHUNK - 2) % 3],
              True, True)
        visit(_NCHUNK - 2, bufs[(_NCHUNK - 2) % 3], bufs[(_NCHUNK - 1) % 3],
              True, True)
        visit(_NCHUNK - 1, bufs[(_NCHUNK - 1) % 3], None, False, False)
        for b in range(3):
            wait_scatters(bufs[b][0], bufs[b][2], bufs[b][4])
        plsc.subcore_barrier()

        pltpu.sync_copy(acc.at[pl.ds(r0, _RPT)], agg_out.at[cid, sid])
        if with_cnt:
            pltpu.sync_copy(cacc.at[pl.ds(sid * _CPT, 640)], zc)
            pltpu.sync_copy(zc, cnt_out.at[cid, 0, pl.ds(sid * 640, 640)])

    return pl.kernel(body, out_type, mesh=mesh, scratch_types=scratch)


def _seg_sum_cnt(*args):
    return _make_seg_sum(True)(*args)


def _seg_sum(*args):
    res = _make_seg_sum(False)(*args)
    return res[0] if isinstance(res, (list, tuple)) else res


def _tc_embed(x, W, b, block_rows):
    M, K = x.shape
    H = W.shape[1]

    def body(x_ref, w_ref, b_ref, o_ref):
        o_ref[...] = (jnp.dot(x_ref[...], w_ref[...],
                              preferred_element_type=jnp.float32) + b_ref[...])

    return pl.pallas_call(
        body,
        grid=(M // block_rows,),
        in_specs=[
            pl.BlockSpec((block_rows, K), lambda i: (i, 0)),
            pl.BlockSpec((K, H), lambda i: (0, 0)),
            pl.BlockSpec((1, H), lambda i: (0, 0)),
        ],
        out_specs=pl.BlockSpec((block_rows, H), lambda i: (i, 0)),
        out_shape=jax.ShapeDtypeStruct((M, H), jnp.float32),
    )(x, W, b.reshape(1, H))


def _tc_sage(agg, cnt, x, Wl, bl, Wr):
    BR = 1000

    def body(a_ref, c_ref, x_ref, wl_ref, bl_ref, wr_ref, o_ref):
        a = a_ref[0] + a_ref[1]
        c = c_ref[:, 0:1] + c_ref[:, 1:2]
        mean = a / jnp.maximum(c, 1.0)
        o_ref[...] = jnp.maximum(
            jnp.dot(mean, wl_ref[...], preferred_element_type=jnp.float32)
            + bl_ref[...]
            + jnp.dot(x_ref[...], wr_ref[...],
                      preferred_element_type=jnp.float32),
            0.0)

    return pl.pallas_call(
        body,
        grid=(_N // BR,),
        in_specs=[
            pl.BlockSpec((_NCORES, BR, _D), lambda i: (0, i, 0)),
            pl.BlockSpec((BR, _NCORES), lambda i: (i, 0)),
            pl.BlockSpec((BR, _D), lambda i: (i, 0)),
            pl.BlockSpec((_D, _D), lambda i: (0, 0)),
            pl.BlockSpec((1, _D), lambda i: (0, 0)),
            pl.BlockSpec((_D, _D), lambda i: (0, 0)),
        ],
        out_specs=pl.BlockSpec((BR, _D), lambda i: (i, 0)),
        out_shape=jax.ShapeDtypeStruct((_N, _D), jnp.float32),
    )(agg, cnt, x, Wl, bl.reshape(1, _D), Wr)


def _tc_head(x0, x1, x2, W1a, W1b, W1c, b1, W2, b2, Wc, bc, Wt, bt, WT_, bT_,
             Woc, boc, Wot, bot, WoT, boT):
    BR = 1000
    HH = 64

    def body(x0r, x1r, x2r, w1ar, w1br, w1cr, b1r, w2r, b2r, wcr, bcr,
             wtr, btr, wTr, bTr, wocr, bocr, wotr, botr, wTor, bTor,
             ot1, ot0, oT, ht1, ht0):
        dot = lambda a, w: jnp.dot(a, w, preferred_element_type=jnp.float32)
        h = jnp.maximum(dot(x0r[...], w1ar[...]) + dot(x1r[...], w1br[...])
                        + dot(x2r[...], w1cr[...]) + b1r[...], 0.0)
        h = jnp.maximum(dot(h, w2r[...]) + b2r[...], 0.0)
        a_t0 = jnp.maximum(dot(h, wcr[...]) + bcr[...], 0.0)
        a_t1 = jnp.maximum(dot(h, wtr[...]) + btr[...], 0.0)
        a_T = jnp.maximum(dot(h, wTr[...]) + bTr[...], 0.0)
        ht0[...] = a_t0
        ht1[...] = a_t1
        ot0[...] = jnp.maximum(dot(a_t0, wocr[...]) + bocr[...], 0.0)[:, :1]
        ot1[...] = jnp.maximum(dot(a_t1, wotr[...]) + botr[...], 0.0)[:, :1]
        oT[...] = jnp.maximum(dot(a_T, wTor[...]) + bTor[...], 0.0)[:, :1]

    full = lambda s: pl.BlockSpec(s, lambda i: tuple(0 for _ in s))
    row_spec = lambda w: pl.BlockSpec((BR, w), lambda i: (i, 0))
    outs = pl.pallas_call(
        body,
        grid=(_NU // BR,),
        in_specs=[
            row_spec(_D), row_spec(_D), row_spec(_D),
            full((_D, _D)), full((_D, _D)), full((_D, _D)), full((1, _D)),
            full((_D, _D)), full((1, _D)),
            full((_D, HH)), full((1, HH)),
            full((_D, HH)), full((1, HH)),
            full((_D, HH)), full((1, HH)),
            full((HH, _D)), full((1, _D)),
            full((HH, _D)), full((1, _D)),
            full((HH, _D)), full((1, _D)),
        ],
        out_specs=[
            row_spec(1), row_spec(1), row_spec(1),
            row_spec(HH), row_spec(HH),
        ],
        out_shape=[
            jax.ShapeDtypeStruct((_NU, 1), jnp.float32),
            jax.ShapeDtypeStruct((_NU, 1), jnp.float32),
            jax.ShapeDtypeStruct((_NU, 1), jnp.float32),
            jax.ShapeDtypeStruct((_NU, HH), jnp.float32),
            jax.ShapeDtypeStruct((_NU, HH), jnp.float32),
        ],
    )(x0, x1, x2, W1a, W1b, W1c, b1.reshape(1, _D), W2, b2.reshape(1, _D),
      Wc, bc.reshape(1, HH), Wt, bt.reshape(1, HH), WT_, bT_.reshape(1, HH),
      Woc, boc.reshape(1, _D), Wot, bot.reshape(1, _D), WoT, boT.reshape(1, _D))
    return outs


def kernel(xu, xp, edge_index, Wu, bu, Wp, bp, Wl0, bl0, Wr0, Wl1, bl1, Wr1,
           Wc1, bc1, Wc2, bc2, Wctl, bctl, Wtrt, btrt, WT, bT, Woc, boc,
           Wot, bot, WoT, boT):
    f32 = jnp.float32
    src = edge_index[0]
    dst = edge_index[1]

    xu_e = _tc_embed(xu, Wu, bu, 1000)
    xp_e = _tc_embed(xp, Wp, bp, 1000)
    emb0 = jnp.concatenate([xu_e, xp_e], axis=0)
    zeros_a = jnp.zeros((_RPT, _D), f32)

    agg0, cntp = _seg_sum_cnt(emb0, src, dst, zeros_a)
    agg0 = agg0.reshape(_NCORES, _N, _D)
    arr = cntp.reshape(_NCORES, _NSUB, 640)
    cnt = jnp.concatenate(
        [arr[:, :_NSUB - 1, :_CPT].reshape(_NCORES, -1), arr[:, _NSUB - 1]],
        axis=1).T
    emb1 = _tc_sage(agg0, cnt, emb0, Wl0, bl0, Wr0)
    agg1 = _seg_sum(emb1, src, dst, zeros_a)
    agg1 = agg1.reshape(_NCORES, _N, _D)
    emb2 = _tc_sage(agg1, cnt, emb1, Wl1, bl1, Wr1)

    pad_w = lambda w: jnp.pad(w, ((0, 0), (0, _D - w.shape[1])))
    pad_b = lambda b: jnp.pad(b, (0, _D - b.shape[0]))

    o_t1, o_t0, o_T, h_t1, h_t0 = _tc_head(
        xu_e, emb1, emb2,
        Wc1[0:_D], Wc1[_D:2 * _D], Wc1[2 * _D:3 * _D], bc1, Wc2, bc2,
        Wctl, bctl, Wtrt, btrt, WT, bT,
        pad_w(Woc), pad_b(boc), pad_w(Wot), pad_b(bot), pad_w(WoT), pad_b(boT))

    return (o_t1, o_t0, o_T, h_t1, h_t0)
